# Initial kernel scaffold; baseline (speedup 1.0000x reference)
import jax
import jax.numpy as jnp
from jax.experimental import pallas as pl


def kernel(boxes, scores, labels):
    raise NotImplementedError("write your pallas kernel here")


# speedup vs baseline: 130.9406x; 130.9406x over previous
import jax
import jax.numpy as jnp
from jax.experimental import pallas as pl
from jax.experimental.pallas import tpu as pltpu

_N = 20000
_IOU_T = 0.7
_SCORE_T = 0.2
_B = 256
_NPAD = 20480
_NT = _NPAD // _B


def _nms_body(x1, y1, x2, y2, s, x1t, y1t, x2t, y2t, keep_ref, a_ref):
    i = pl.program_id(0)

    @pl.when(i == 0)
    def _init():
        a_ref[...] = (x2[...] - x1[...]) * (y2[...] - y1[...])
        keep_ref[...] = jnp.where(s[...] >= _SCORE_T, 1.0, 0.0)

    xi1 = x1t[...].reshape(_B, 1)
    yi1 = y1t[...].reshape(_B, 1)
    xi2 = x2t[...].reshape(_B, 1)
    yi2 = y2t[...].reshape(_B, 1)
    ai = (xi2 - xi1) * (yi2 - yi1)

    def _hits(j):
        xj1 = x1[pl.ds(j, 1), :]
        yj1 = y1[pl.ds(j, 1), :]
        xj2 = x2[pl.ds(j, 1), :]
        yj2 = y2[pl.ds(j, 1), :]
        aj = a_ref[pl.ds(j, 1), :]
        xx1 = jnp.maximum(xi1, xj1)
        yy1 = jnp.maximum(yi1, yj1)
        xx2 = jnp.minimum(xi2, xj2)
        yy2 = jnp.minimum(yi2, yj2)
        inter = jnp.maximum(xx2 - xx1, 0.0) * jnp.maximum(yy2 - yy1, 0.0)
        iou = inter / (ai + aj - inter + 1e-9)
        return iou > _IOU_T

    row = jax.lax.broadcasted_iota(jnp.int32, (_B, _B), 0)
    col = jax.lax.broadcasted_iota(jnp.int32, (_B, _B), 1)
    m_self = jnp.where(_hits(i) & (row < col), 1.0, 0.0)
    alive0 = keep_ref[pl.ds(i, 1), :]

    def _wcond(c):
        return c[1]

    def _wbody(c):
        kept, _ = c
        sup = jax.lax.dot_general(
            kept, m_self, (((1,), (0,)), ((), ())),
            preferred_element_type=jnp.float32)
        new = jnp.where(sup > 0.5, 0.0, alive0)
        return new, jnp.any(new != kept)

    kept, _ = jax.lax.while_loop(_wcond, _wbody, (alive0, True))
    keep_ref[pl.ds(i, 1), :] = kept

    def _inner(j, carry):
        hits = jnp.where(_hits(j), 1.0, 0.0)
        dead = jax.lax.dot_general(
            kept, hits, (((1,), (0,)), ((), ())),
            preferred_element_type=jnp.float32)
        kj = keep_ref[pl.ds(j, 1), :]
        keep_ref[pl.ds(j, 1), :] = jnp.where(dead > 0.5, 0.0, kj)
        return carry

    jax.lax.fori_loop(i + 1, _NT, _inner, 0)


def _nms_keep_mask(bp, sp):
    x1 = bp[:, 0].reshape(_NT, _B)
    y1 = bp[:, 1].reshape(_NT, _B)
    x2 = bp[:, 2].reshape(_NT, _B)
    y2 = bp[:, 3].reshape(_NT, _B)
    s2 = sp.reshape(_NT, _B)
    x1t = x1.reshape(_NT, _B, 1)
    y1t = y1.reshape(_NT, _B, 1)
    x2t = x2.reshape(_NT, _B, 1)
    y2t = y2.reshape(_NT, _B, 1)

    full = pl.BlockSpec((_NT, _B), lambda i: (0, 0))
    colspec = pl.BlockSpec((1, _B, 1), lambda i: (i, 0, 0))
    keep = pl.pallas_call(
        _nms_body,
        grid=(_NT,),
        in_specs=[full, full, full, full, full,
                  colspec, colspec, colspec, colspec],
        out_specs=full,
        out_shape=jax.ShapeDtypeStruct((_NT, _B), jnp.float32),
        scratch_shapes=[pltpu.VMEM((_NT, _B), jnp.float32)],
    )(x1, y1, x2, y2, s2, x1t, y1t, x2t, y2t)
    return keep.reshape(_NPAD)[:_N]


@jax.jit
def kernel(boxes, scores, labels):
    max_coord = boxes.max()
    offsets = labels.astype(boxes.dtype) * (max_coord + 1.0)
    b_all = boxes + offsets[:, None]
    order = jnp.argsort(-scores)
    b = b_all[order]
    s = scores[order]
    pad = _NPAD - _N
    bp = jnp.pad(b, ((0, pad), (0, 0)))
    sp = jnp.pad(s, ((0, pad),), constant_values=-1.0)
    keep_sorted = _nms_keep_mask(bp, sp)
    m = jnp.zeros((_N,), jnp.float32).at[order].set(keep_sorted)
    out = jnp.concatenate([boxes * m[:, None], (scores * m)[:, None]], axis=1)
    return out


# speedup vs baseline: 221.0275x; 1.6880x over previous
import jax
import jax.numpy as jnp
from jax.experimental import pallas as pl
from jax.experimental.pallas import tpu as pltpu

_N = 20000
_NUM_CLASSES = 80
_IOU_T = 0.7
_SCORE_T = 0.2
_B = 256
_NPAD = 20480
_NT = _NPAD // _B


def _nms_body(jt, x1, y1, x2, y2, s, x1t, y1t, x2t, y2t, keep_ref, a_ref):
    i = pl.program_id(0)

    @pl.when(i == 0)
    def _init():
        a_ref[...] = (x2[...] - x1[...]) * (y2[...] - y1[...])
        keep_ref[...] = jnp.where(s[...] >= _SCORE_T, 1.0, 0.0)

    xi1 = x1t[...].reshape(_B, 1)
    yi1 = y1t[...].reshape(_B, 1)
    xi2 = x2t[...].reshape(_B, 1)
    yi2 = y2t[...].reshape(_B, 1)
    ai = (xi2 - xi1) * (yi2 - yi1)

    def _hits(j):
        xj1 = x1[pl.ds(j, 1), :]
        yj1 = y1[pl.ds(j, 1), :]
        xj2 = x2[pl.ds(j, 1), :]
        yj2 = y2[pl.ds(j, 1), :]
        aj = a_ref[pl.ds(j, 1), :]
        xx1 = jnp.maximum(xi1, xj1)
        yy1 = jnp.maximum(yi1, yj1)
        xx2 = jnp.minimum(xi2, xj2)
        yy2 = jnp.minimum(yi2, yj2)
        inter = jnp.maximum(xx2 - xx1, 0.0) * jnp.maximum(yy2 - yy1, 0.0)
        iou = inter / (ai + aj - inter + 1e-9)
        return iou > _IOU_T

    row = jax.lax.broadcasted_iota(jnp.int32, (_B, _B), 0)
    col = jax.lax.broadcasted_iota(jnp.int32, (_B, _B), 1)
    m_self = jnp.where(_hits(i) & (row < col), 1.0, 0.0)
    alive0 = keep_ref[pl.ds(i, 1), :]

    def _wcond(c):
        return c[1]

    def _wbody(c):
        kept, _ = c
        sup = jax.lax.dot_general(
            kept, m_self, (((1,), (0,)), ((), ())),
            preferred_element_type=jnp.float32)
        new = jnp.where(sup > 0.5, 0.0, alive0)
        return new, jnp.any(new != kept)

    kept, _ = jax.lax.while_loop(_wcond, _wbody, (alive0, True))
    keep_ref[pl.ds(i, 1), :] = kept

    def _inner(j, carry):
        hits = jnp.where(_hits(j), 1.0, 0.0)
        dead = jax.lax.dot_general(
            kept, hits, (((1,), (0,)), ((), ())),
            preferred_element_type=jnp.float32)
        kj = keep_ref[pl.ds(j, 1), :]
        keep_ref[pl.ds(j, 1), :] = jnp.where(dead > 0.5, 0.0, kj)
        return carry

    jax.lax.fori_loop(i + 1, jt[i], _inner, 0)


def _nms_keep_mask(bp, sp, jt_end):
    x1 = bp[:, 0].reshape(_NT, _B)
    y1 = bp[:, 1].reshape(_NT, _B)
    x2 = bp[:, 2].reshape(_NT, _B)
    y2 = bp[:, 3].reshape(_NT, _B)
    s2 = sp.reshape(_NT, _B)
    x1t = x1.reshape(_NT, _B, 1)
    y1t = y1.reshape(_NT, _B, 1)
    x2t = x2.reshape(_NT, _B, 1)
    y2t = y2.reshape(_NT, _B, 1)

    full = pl.BlockSpec((_NT, _B), lambda i: (0, 0))
    colspec = pl.BlockSpec((1, _B, 1), lambda i: (i, 0, 0))
    smem = pl.BlockSpec(memory_space=pltpu.SMEM)
    keep = pl.pallas_call(
        _nms_body,
        grid=(_NT,),
        in_specs=[smem, full, full, full, full, full,
                  colspec, colspec, colspec, colspec],
        out_specs=full,
        out_shape=jax.ShapeDtypeStruct((_NT, _B), jnp.float32),
        scratch_shapes=[pltpu.VMEM((_NT, _B), jnp.float32)],
    )(jt_end, x1, y1, x2, y2, s2, x1t, y1t, x2t, y2t)
    return keep.reshape(_NPAD)[:_N]


@jax.jit
def kernel(boxes, scores, labels):
    max_coord = boxes.max()
    offsets = labels.astype(boxes.dtype) * (max_coord + 1.0)
    b_all = boxes + offsets[:, None]
    order0 = jnp.argsort(-scores)
    order = order0[jnp.argsort(labels[order0], stable=True)]
    b = b_all[order]
    s = scores[order]
    counts = jnp.bincount(labels, length=_NUM_CLASSES)
    ends = jnp.cumsum(counts).astype(jnp.int32)
    seg_end = ends[labels[order]]
    pad = _NPAD - _N
    seg_end = jnp.pad(seg_end, ((0, pad),))
    jt_end = (seg_end.reshape(_NT, _B).max(axis=1) + _B - 1) // _B
    pad = _NPAD - _N
    bp = jnp.pad(b, ((0, pad), (0, 0)))
    sp = jnp.pad(s, ((0, pad),), constant_values=-1.0)
    keep_sorted = _nms_keep_mask(bp, sp, jt_end)
    m = jnp.zeros((_N,), jnp.float32).at[order].set(keep_sorted)
    out = jnp.concatenate([boxes * m[:, None], (scores * m)[:, None]], axis=1)
    return out


# speedup vs baseline: 237.0257x; 1.0724x over previous
import jax
import jax.numpy as jnp
from jax.experimental import pallas as pl
from jax.experimental.pallas import tpu as pltpu

_N = 20000
_NUM_CLASSES = 80
_IOU_T = 0.7
_SCORE_T = 0.2
_B = 256
_NPAD = 20480
_NT = _NPAD // _B


def _nms_body(jt, x1, y1, x2, y2, s, x1t, y1t, x2t, y2t, keep_ref, a_ref):
    i = pl.program_id(0)

    @pl.when(i == 0)
    def _init():
        a_ref[...] = (x2[...] - x1[...]) * (y2[...] - y1[...])
        keep_ref[...] = jnp.where(s[...] >= _SCORE_T, 1.0, 0.0)

    xi1 = x1t[...].reshape(_B, 1)
    yi1 = y1t[...].reshape(_B, 1)
    xi2 = x2t[...].reshape(_B, 1)
    yi2 = y2t[...].reshape(_B, 1)
    ai = (xi2 - xi1) * (yi2 - yi1)

    def _hits(j):
        xj1 = x1[pl.ds(j, 1), :]
        yj1 = y1[pl.ds(j, 1), :]
        xj2 = x2[pl.ds(j, 1), :]
        yj2 = y2[pl.ds(j, 1), :]
        aj = a_ref[pl.ds(j, 1), :]
        xx1 = jnp.maximum(xi1, xj1)
        yy1 = jnp.maximum(yi1, yj1)
        xx2 = jnp.minimum(xi2, xj2)
        yy2 = jnp.minimum(yi2, yj2)
        inter = jnp.maximum(xx2 - xx1, 0.0) * jnp.maximum(yy2 - yy1, 0.0)
        iou = inter / (ai + aj - inter + 1e-9)
        return iou > _IOU_T

    row = jax.lax.broadcasted_iota(jnp.int32, (_B, _B), 0)
    col = jax.lax.broadcasted_iota(jnp.int32, (_B, _B), 1)
    m_self = jnp.where(_hits(i) & (row < col), 1.0, 0.0)
    alive0 = keep_ref[pl.ds(i, 1), :]

    def _wcond(c):
        return c[1]

    def _wbody(c):
        kept, _ = c
        sup = jax.lax.dot_general(
            kept, m_self, (((1,), (0,)), ((), ())),
            preferred_element_type=jnp.float32)
        new = jnp.where(sup > 0.5, 0.0, alive0)
        return new, jnp.any(new != kept)

    kept, _ = jax.lax.while_loop(_wcond, _wbody, (alive0, True))
    keep_ref[pl.ds(i, 1), :] = kept

    def _inner(j, carry):
        hits = jnp.where(_hits(j), 1.0, 0.0)
        dead = jax.lax.dot_general(
            kept, hits, (((1,), (0,)), ((), ())),
            preferred_element_type=jnp.float32)
        kj = keep_ref[pl.ds(j, 1), :]
        keep_ref[pl.ds(j, 1), :] = jnp.where(dead > 0.5, 0.0, kj)
        return carry

    jax.lax.fori_loop(i + 1, jt[i], _inner, 0)


def _nms_keep_mask(bp, sp, jt_end):
    x1 = bp[:, 0].reshape(_NT, _B)
    y1 = bp[:, 1].reshape(_NT, _B)
    x2 = bp[:, 2].reshape(_NT, _B)
    y2 = bp[:, 3].reshape(_NT, _B)
    s2 = sp.reshape(_NT, _B)
    x1t = x1.reshape(_NT, _B, 1)
    y1t = y1.reshape(_NT, _B, 1)
    x2t = x2.reshape(_NT, _B, 1)
    y2t = y2.reshape(_NT, _B, 1)

    full = pl.BlockSpec((_NT, _B), lambda i: (0, 0))
    colspec = pl.BlockSpec((1, _B, 1), lambda i: (i, 0, 0))
    smem = pl.BlockSpec(memory_space=pltpu.SMEM)
    keep = pl.pallas_call(
        _nms_body,
        grid=(_NT,),
        in_specs=[smem, full, full, full, full, full,
                  colspec, colspec, colspec, colspec],
        out_specs=full,
        out_shape=jax.ShapeDtypeStruct((_NT, _B), jnp.float32),
        scratch_shapes=[pltpu.VMEM((_NT, _B), jnp.float32)],
    )(jt_end, x1, y1, x2, y2, s2, x1t, y1t, x2t, y2t)
    return keep.reshape(_NPAD)[:_N]


@jax.jit
def kernel(boxes, scores, labels):
    max_coord = boxes.max()
    offsets = labels.astype(boxes.dtype) * (max_coord + 1.0)
    b_all = boxes + offsets[:, None]
    idx = jnp.arange(_N, dtype=jnp.int32)
    ls, _, order = jax.lax.sort((labels, -scores, idx), num_keys=2,
                                is_stable=True)
    b = b_all[order]
    s = scores[order]
    counts = jnp.bincount(labels, length=_NUM_CLASSES)
    ends = jnp.cumsum(counts).astype(jnp.int32)
    seg_end = ends[ls]
    pad = _NPAD - _N
    seg_end = jnp.pad(seg_end, ((0, pad),))
    jt_end = (seg_end.reshape(_NT, _B).max(axis=1) + _B - 1) // _B
    pad = _NPAD - _N
    bp = jnp.pad(b, ((0, pad), (0, 0)))
    sp = jnp.pad(s, ((0, pad),), constant_values=-1.0)
    keep_sorted = _nms_keep_mask(bp, sp, jt_end)
    m = jnp.zeros((_N,), jnp.float32).at[order].set(keep_sorted)
    out = jnp.concatenate([boxes * m[:, None], (scores * m)[:, None]], axis=1)
    return out


# speedup vs baseline: 257.2127x; 1.0852x over previous
import jax
import jax.numpy as jnp
from jax.experimental import pallas as pl
from jax.experimental.pallas import tpu as pltpu

_N = 20000
_NUM_CLASSES = 80
_IOU_T = 0.7
_SCORE_T = 0.2
_B = 256
_NPAD = 20480
_NT = _NPAD // _B


def _nms_body(jt, x1, y1, x2, y2, s, x1t, y1t, x2t, y2t, keep_ref, a_ref):
    a_ref[...] = (x2[...] - x1[...]) * (y2[...] - y1[...])
    keep_ref[...] = jnp.where(s[...] >= _SCORE_T, 1.0, 0.0)

    lane = jax.lax.broadcasted_iota(jnp.int32, (1, _NT), 1)
    row = jax.lax.broadcasted_iota(jnp.int32, (_B, _B), 0)
    col = jax.lax.broadcasted_iota(jnp.int32, (_B, _B), 1)
    upper = row < col

    def _outer(i, carry):
        oh = jnp.where(lane == i, 1.0, 0.0)
        xi1 = jnp.sum(x1t[...] * oh, axis=1, keepdims=True)
        yi1 = jnp.sum(y1t[...] * oh, axis=1, keepdims=True)
        xi2 = jnp.sum(x2t[...] * oh, axis=1, keepdims=True)
        yi2 = jnp.sum(y2t[...] * oh, axis=1, keepdims=True)
        ai = (xi2 - xi1) * (yi2 - yi1)

        def _hits(j):
            xj1 = x1[pl.ds(j, 1), :]
            yj1 = y1[pl.ds(j, 1), :]
            xj2 = x2[pl.ds(j, 1), :]
            yj2 = y2[pl.ds(j, 1), :]
            aj = a_ref[pl.ds(j, 1), :]
            xx1 = jnp.maximum(xi1, xj1)
            yy1 = jnp.maximum(yi1, yj1)
            xx2 = jnp.minimum(xi2, xj2)
            yy2 = jnp.minimum(yi2, yj2)
            inter = jnp.maximum(xx2 - xx1, 0.0) * jnp.maximum(yy2 - yy1, 0.0)
            iou = inter / (ai + aj - inter + 1e-9)
            return iou > _IOU_T

        m_self = jnp.where(_hits(i) & upper, 1.0, 0.0)
        alive0 = keep_ref[pl.ds(i, 1), :]

        def _wcond(c):
            return c[1]

        def _wbody(c):
            kept, _ = c
            sup = jax.lax.dot_general(
                kept, m_self, (((1,), (0,)), ((), ())),
                preferred_element_type=jnp.float32)
            new = jnp.where(sup > 0.5, 0.0, alive0)
            return new, jnp.any(new != kept)

        kept, _ = jax.lax.while_loop(_wcond, _wbody, (alive0, True))
        keep_ref[pl.ds(i, 1), :] = kept

        def _inner(j, c):
            hits = jnp.where(_hits(j), 1.0, 0.0)
            dead = jax.lax.dot_general(
                kept, hits, (((1,), (0,)), ((), ())),
                preferred_element_type=jnp.float32)
            kj = keep_ref[pl.ds(j, 1), :]
            keep_ref[pl.ds(j, 1), :] = jnp.where(dead > 0.5, 0.0, kj)
            return c

        jax.lax.fori_loop(i + 1, jt[i], _inner, 0)
        return carry

    jax.lax.fori_loop(0, _NT, _outer, 0)


def _nms_keep_mask(bp, sp, jt_end):
    x1 = bp[:, 0].reshape(_NT, _B)
    y1 = bp[:, 1].reshape(_NT, _B)
    x2 = bp[:, 2].reshape(_NT, _B)
    y2 = bp[:, 3].reshape(_NT, _B)
    s2 = sp.reshape(_NT, _B)
    x1t = x1.T
    y1t = y1.T
    x2t = x2.T
    y2t = y2.T

    vmem = pl.BlockSpec(memory_space=pltpu.VMEM)
    smem = pl.BlockSpec(memory_space=pltpu.SMEM)
    keep = pl.pallas_call(
        _nms_body,
        in_specs=[smem] + [vmem] * 9,
        out_specs=vmem,
        out_shape=jax.ShapeDtypeStruct((_NT, _B), jnp.float32),
        scratch_shapes=[pltpu.VMEM((_NT, _B), jnp.float32)],
    )(jt_end, x1, y1, x2, y2, s2, x1t, y1t, x2t, y2t)
    return keep.reshape(_NPAD)[:_N]


@jax.jit
def kernel(boxes, scores, labels):
    max_coord = boxes.max()
    offsets = labels.astype(boxes.dtype) * (max_coord + 1.0)
    b_all = boxes + offsets[:, None]
    idx = jnp.arange(_N, dtype=jnp.int32)
    ls, _, order = jax.lax.sort((labels, -scores, idx), num_keys=2,
                                is_stable=True)
    b = b_all[order]
    s = scores[order]
    counts = jnp.bincount(labels, length=_NUM_CLASSES)
    ends = jnp.cumsum(counts).astype(jnp.int32)
    seg_end = ends[ls]
    pad = _NPAD - _N
    seg_end = jnp.pad(seg_end, ((0, pad),))
    jt_end = (seg_end.reshape(_NT, _B).max(axis=1) + _B - 1) // _B
    bp = jnp.pad(b, ((0, pad), (0, 0)))
    sp = jnp.pad(s, ((0, pad),), constant_values=-1.0)
    keep_sorted = _nms_keep_mask(bp, sp, jt_end)
    m = jnp.zeros((_N,), jnp.float32).at[order].set(keep_sorted)
    out = jnp.concatenate([boxes * m[:, None], (scores * m)[:, None]], axis=1)
    return out


# speedup vs baseline: 259.1295x; 1.0075x over previous
import jax
import jax.numpy as jnp
from jax.experimental import pallas as pl
from jax.experimental.pallas import tpu as pltpu

_N = 20000
_NUM_CLASSES = 80
_IOU_T = 0.7
_SCORE_T = 0.2
_B = 512
_NPAD = 20480
_NT = _NPAD // _B


def _nms_body(jt, x1, y1, x2, y2, s, x1t, y1t, x2t, y2t, keep_ref, a_ref):
    a_ref[...] = (x2[...] - x1[...]) * (y2[...] - y1[...])
    keep_ref[...] = jnp.where(s[...] >= _SCORE_T, 1.0, 0.0)

    lane = jax.lax.broadcasted_iota(jnp.int32, (1, _NT), 1)
    row = jax.lax.broadcasted_iota(jnp.int32, (_B, _B), 0)
    col = jax.lax.broadcasted_iota(jnp.int32, (_B, _B), 1)
    upper = row < col

    def _outer(i, carry):
        oh = jnp.where(lane == i, 1.0, 0.0)
        xi1 = jnp.sum(x1t[...] * oh, axis=1, keepdims=True)
        yi1 = jnp.sum(y1t[...] * oh, axis=1, keepdims=True)
        xi2 = jnp.sum(x2t[...] * oh, axis=1, keepdims=True)
        yi2 = jnp.sum(y2t[...] * oh, axis=1, keepdims=True)
        ai = (xi2 - xi1) * (yi2 - yi1)

        def _hits(j):
            xj1 = x1[pl.ds(j, 1), :]
            yj1 = y1[pl.ds(j, 1), :]
            xj2 = x2[pl.ds(j, 1), :]
            yj2 = y2[pl.ds(j, 1), :]
            aj = a_ref[pl.ds(j, 1), :]
            xx1 = jnp.maximum(xi1, xj1)
            yy1 = jnp.maximum(yi1, yj1)
            xx2 = jnp.minimum(xi2, xj2)
            yy2 = jnp.minimum(yi2, yj2)
            inter = jnp.maximum(xx2 - xx1, 0.0) * jnp.maximum(yy2 - yy1, 0.0)
            iou = inter / (ai + aj - inter + 1e-9)
            return iou > _IOU_T

        m_self = jnp.where(_hits(i) & upper, 1.0, 0.0)
        alive0 = keep_ref[pl.ds(i, 1), :]

        def _wcond(c):
            return c[1]

        def _wbody(c):
            kept, _ = c
            sup = jax.lax.dot_general(
                kept, m_self, (((1,), (0,)), ((), ())),
                preferred_element_type=jnp.float32)
            new = jnp.where(sup > 0.5, 0.0, alive0)
            return new, jnp.any(new != kept)

        kept, _ = jax.lax.while_loop(_wcond, _wbody, (alive0, True))
        keep_ref[pl.ds(i, 1), :] = kept

        def _inner(j, c):
            hits = jnp.where(_hits(j), 1.0, 0.0)
            dead = jax.lax.dot_general(
                kept, hits, (((1,), (0,)), ((), ())),
                preferred_element_type=jnp.float32)
            kj = keep_ref[pl.ds(j, 1), :]
            keep_ref[pl.ds(j, 1), :] = jnp.where(dead > 0.5, 0.0, kj)
            return c

        jax.lax.fori_loop(i + 1, jt[i], _inner, 0)
        return carry

    jax.lax.fori_loop(0, _NT, _outer, 0)


def _nms_keep_mask(bp, sp, jt_end):
    x1 = bp[:, 0].reshape(_NT, _B)
    y1 = bp[:, 1].reshape(_NT, _B)
    x2 = bp[:, 2].reshape(_NT, _B)
    y2 = bp[:, 3].reshape(_NT, _B)
    s2 = sp.reshape(_NT, _B)
    x1t = x1.T
    y1t = y1.T
    x2t = x2.T
    y2t = y2.T

    vmem = pl.BlockSpec(memory_space=pltpu.VMEM)
    smem = pl.BlockSpec(memory_space=pltpu.SMEM)
    keep = pl.pallas_call(
        _nms_body,
        in_specs=[smem] + [vmem] * 9,
        out_specs=vmem,
        out_shape=jax.ShapeDtypeStruct((_NT, _B), jnp.float32),
        scratch_shapes=[pltpu.VMEM((_NT, _B), jnp.float32)],
    )(jt_end, x1, y1, x2, y2, s2, x1t, y1t, x2t, y2t)
    return keep.reshape(_NPAD)[:_N]


@jax.jit
def kernel(boxes, scores, labels):
    max_coord = boxes.max()
    offsets = labels.astype(boxes.dtype) * (max_coord + 1.0)
    b_all = boxes + offsets[:, None]
    idx = jnp.arange(_N, dtype=jnp.int32)
    ls, _, order = jax.lax.sort((labels, -scores, idx), num_keys=2,
                                is_stable=True)
    b = b_all[order]
    s = scores[order]
    counts = jnp.bincount(labels, length=_NUM_CLASSES)
    ends = jnp.cumsum(counts).astype(jnp.int32)
    seg_end = ends[ls]
    pad = _NPAD - _N
    seg_end = jnp.pad(seg_end, ((0, pad),))
    jt_end = (seg_end.reshape(_NT, _B).max(axis=1) + _B - 1) // _B
    bp = jnp.pad(b, ((0, pad), (0, 0)))
    sp = jnp.pad(s, ((0, pad),), constant_values=-1.0)
    keep_sorted = _nms_keep_mask(bp, sp, jt_end)
    m = jnp.zeros((_N,), jnp.float32).at[order].set(keep_sorted)
    out = jnp.concatenate([boxes * m[:, None], (scores * m)[:, None]], axis=1)
    return out


# speedup vs baseline: 439.0884x; 1.6945x over previous
import jax
import jax.numpy as jnp
from jax.experimental import pallas as pl
from jax.experimental.pallas import tpu as pltpu

_N = 20000
_NUM_CLASSES = 80
_IOU_T = 0.7
_SCORE_T = 0.2
_B = 512
_NPAD = 20480
_NT = _NPAD // _B


def _nms_body(jt, x1, y1, x2, y2, s, keep_ref, a_ref):
    a_ref[...] = (x2[...] - x1[...]) * (y2[...] - y1[...])
    keep_ref[...] = jnp.where(s[...] >= _SCORE_T, 1.0, 0.0)

    row = jax.lax.broadcasted_iota(jnp.int32, (_B, _B), 0)
    col = jax.lax.broadcasted_iota(jnp.int32, (_B, _B), 1)
    upper = row < col

    def _outer(i, carry):
        xi1 = x1[pl.ds(i, 1), :].T
        yi1 = y1[pl.ds(i, 1), :].T
        xi2 = x2[pl.ds(i, 1), :].T
        yi2 = y2[pl.ds(i, 1), :].T
        ai = (xi2 - xi1) * (yi2 - yi1)

        def _hits(j):
            xj1 = x1[pl.ds(j, 1), :]
            yj1 = y1[pl.ds(j, 1), :]
            xj2 = x2[pl.ds(j, 1), :]
            yj2 = y2[pl.ds(j, 1), :]
            aj = a_ref[pl.ds(j, 1), :]
            xx1 = jnp.maximum(xi1, xj1)
            yy1 = jnp.maximum(yi1, yj1)
            xx2 = jnp.minimum(xi2, xj2)
            yy2 = jnp.minimum(yi2, yj2)
            inter = jnp.maximum(xx2 - xx1, 0.0) * jnp.maximum(yy2 - yy1, 0.0)
            iou = inter / (ai + aj - inter + 1e-9)
            return iou > _IOU_T

        m_self = jnp.where(_hits(i) & upper, 1.0, 0.0)
        alive0 = keep_ref[pl.ds(i, 1), :]

        def _wcond(c):
            return c[1]

        def _wbody(c):
            kept, _ = c
            sup = jax.lax.dot_general(
                kept, m_self, (((1,), (0,)), ((), ())),
                preferred_element_type=jnp.float32)
            new = jnp.where(sup > 0.5, 0.0, alive0)
            return new, jnp.any(new != kept)

        kept, _ = jax.lax.while_loop(_wcond, _wbody, (alive0, True))
        keep_ref[pl.ds(i, 1), :] = kept

        def _inner(j, c):
            hits = jnp.where(_hits(j), 1.0, 0.0)
            dead = jax.lax.dot_general(
                kept, hits, (((1,), (0,)), ((), ())),
                preferred_element_type=jnp.float32)
            kj = keep_ref[pl.ds(j, 1), :]
            keep_ref[pl.ds(j, 1), :] = jnp.where(dead > 0.5, 0.0, kj)
            return c

        jax.lax.fori_loop(i + 1, jt[i], _inner, 0)
        return carry

    jax.lax.fori_loop(0, _NT, _outer, 0)


def _nms_keep_mask(cols, sp, jt_end):
    x1, y1, x2, y2 = (c.reshape(_NT, _B) for c in cols)
    s2 = sp.reshape(_NT, _B)

    vmem = pl.BlockSpec(memory_space=pltpu.VMEM)
    smem = pl.BlockSpec(memory_space=pltpu.SMEM)
    keep = pl.pallas_call(
        _nms_body,
        in_specs=[smem] + [vmem] * 5,
        out_specs=vmem,
        out_shape=jax.ShapeDtypeStruct((_NT, _B), jnp.float32),
        scratch_shapes=[pltpu.VMEM((_NT, _B), jnp.float32)],
    )(jt_end, x1, y1, x2, y2, s2)
    return keep.reshape(_NPAD)[:_N]


@jax.jit
def kernel(boxes, scores, labels):
    max_coord = boxes.max()
    offsets = labels.astype(boxes.dtype) * (max_coord + 1.0)
    idx = jnp.arange(_N, dtype=jnp.int32)
    ls, negs, c0, c1, c2, c3, order = jax.lax.sort(
        (labels, -scores,
         boxes[:, 0] + offsets, boxes[:, 1] + offsets,
         boxes[:, 2] + offsets, boxes[:, 3] + offsets,
         idx),
        num_keys=2, is_stable=True)
    s = -negs
    cols = [c0, c1, c2, c3]
    pos = jnp.arange(_N, dtype=jnp.int32)
    is_end = jnp.concatenate([ls[1:] != ls[:-1],
                              jnp.ones((1,), dtype=bool)])
    seg_end = jax.lax.cummin(jnp.where(is_end, pos + 1, _N), reverse=True)
    pad = _NPAD - _N
    seg_end = jnp.pad(seg_end, ((0, pad),))
    jt_end = (seg_end.reshape(_NT, _B).max(axis=1) + _B - 1) // _B
    cols = [jnp.pad(c, ((0, pad),)) for c in cols]
    sp = jnp.pad(s, ((0, pad),), constant_values=-1.0)
    keep_sorted = _nms_keep_mask(cols, sp, jt_end)
    m = jnp.zeros((_N,), jnp.float32).at[order].set(keep_sorted)
    out = jnp.concatenate([boxes * m[:, None], (scores * m)[:, None]], axis=1)
    return out
